# baseline (speedup 1.0000x reference)
import functools

import jax
import jax.numpy as jnp
from jax import lax
from jax.experimental import pallas as pl
from jax.experimental.pallas import tpu as pltpu
from jax.experimental.pallas import tpu_sc as plsc

_DEF = jax.lax.Precision.DEFAULT
_HI = jax.lax.Precision.HIGHEST


def _tc_body(kl_ref, ts_ref, t_ref, rcp_ref, tso_ref):
    mt = jnp.mean(kl_ref[...], axis=0)
    m = jnp.transpose(mt, (1, 0))
    m = m / jnp.sqrt(jnp.sum(m * m, axis=1, keepdims=True))
    ci = jax.lax.broadcasted_iota(jnp.int32, (1024, 2048), 1)
    rk = jax.lax.broadcasted_iota(jnp.int32, (1024, 1), 0)
    so = jnp.where(ci == 2 * rk + 1, 1.0, 0.0)
    b = jax.lax.dot_general(so, m, (((1,), (0,)), ((), ())),
                            precision=_HI)
    scores = jax.lax.dot_general(m, b, (((1,), (1,)), ((), ())),
                                 precision=_DEF)
    node_max = jnp.max(scores, axis=1, keepdims=True)
    ck = jax.lax.broadcasted_iota(jnp.int32, (2048, 1024), 1)
    nidx = jnp.min(jnp.where(scores == node_max, ck, 1024),
                   axis=1, keepdims=True)
    ri = jax.lax.broadcasted_iota(jnp.int32, (2048, 1), 0)
    evenok = (ri % 2 == 0) & (ri >= 2)
    merge = evenok & (ck == nidx)
    is_dst = ri == 2 * ck + 1
    mm = jnp.where(merge | is_dst, 1.0, 0.0)
    ones = jnp.ones((2048, 1), dtype=jnp.float32)
    cnt1 = jax.lax.dot_general(mm, ones, (((0,), (0,)), ((), ())),
                               precision=_DEF)
    ts_y = jax.lax.dot_general(mm, ts_ref[...], (((0,), (0,)), ((), ())),
                               precision=_DEF)
    odd = ri % 2 == 1
    t_ref[...] = jnp.where(odd, 1 + (ri - 1) // 2,
                           jnp.where(ri == 0, 0, 1 + nidx))
    rcp_ref[...] = 1.0 / cnt1
    tso_ref[0:1, :] = ts_ref[0:1, :]
    tso_ref[pl.ds(1, 1024), :] = ts_y


def _sc_body(hid_hbm, idx_hbm, rcp_hbm, out_hbm, idx_v, rcp_v, src_v, acc):
    c = lax.axis_index("c")
    s = lax.axis_index("s")
    t = s * 2 + c

    @pl.when(t < 24)
    def _():
        tc = t // 4
        lc0 = 32 * (t % 4)
        c0 = t * 32
        pltpu.sync_copy(idx_hbm, idx_v)
        pltpu.sync_copy(rcp_hbm, rcp_v)
        pltpu.sync_copy(hid_hbm.at[:, tc, :, pl.ds(lc0, 32)], src_v)

        def _zero(g, _):
            z = jnp.zeros((16,), dtype=jnp.float32)
            acc[g, pl.ds(0, 16)] = z
            acc[g, pl.ds(16, 16)] = z
            return 0

        lax.fori_loop(0, 1026, _zero, 0)

        def _accum(g, _):
            idx16 = idx_v[pl.ds(g * 16, 16)]
            for r in range(16):
                d = idx16[r]
                tr = 2 * g + r // 8
                sr = r % 8
                acc[d, pl.ds(0, 16)] = (
                    acc[d, pl.ds(0, 16)] + src_v[tr, sr, pl.ds(0, 16)])
                acc[d, pl.ds(16, 16)] = (
                    acc[d, pl.ds(16, 16)] + src_v[tr, sr, pl.ds(16, 16)])
            return 0

        lax.fori_loop(0, 128, _accum, 0)

        def _scale(g, _):
            w16 = rcp_v[pl.ds(g * 16, 16)]
            for r in range(16):
                row = 1 + g * 16 + r
                w = jnp.full((16,), w16[r], dtype=jnp.float32)
                acc[row, pl.ds(0, 16)] = acc[row, pl.ds(0, 16)] * w
                acc[row, pl.ds(16, 16)] = acc[row, pl.ds(16, 16)] * w
            return 0

        lax.fori_loop(0, 64, _scale, 0)
        pltpu.sync_copy(acc.at[pl.ds(0, 1025)], out_hbm.at[:, pl.ds(c0, 32)])


_sc_merge = functools.partial(
    pl.kernel,
    out_type=jax.ShapeDtypeStruct((1025, 768), jnp.float32),
    mesh=plsc.VectorSubcoreMesh(core_axis_name="c", subcore_axis_name="s"),
    compiler_params=pltpu.CompilerParams(use_tc_tiling_on_sc=False),
    scratch_types=[
        pltpu.VMEM((2048,), jnp.int32),
        pltpu.VMEM((1024,), jnp.float32),
        pltpu.VMEM((256, 8, 32), jnp.float32),
        pltpu.VMEM((1026, 32), jnp.float32),
    ],
)(_sc_body)


def kernel(hidden_states, attention_mask, self_attention_scores, key_layer,
           tome_size):
    del attention_mask, self_attention_scores
    tgt, rcp, ts_out = pl.pallas_call(
        _tc_body,
        out_shape=(
            jax.ShapeDtypeStruct((2048, 1), jnp.int32),
            jax.ShapeDtypeStruct((1024, 1), jnp.float32),
            jax.ShapeDtypeStruct((1025, 1), jnp.float32),
        ),
    )(jnp.transpose(key_layer[0], (0, 2, 1)), tome_size[0])

    hid4 = jnp.transpose(hidden_states[0].reshape(256, 8, 6, 128),
                         (0, 2, 1, 3))
    out = _sc_merge(hid4, tgt.reshape(2048), rcp.reshape(1024))

    preserved = out[None]
    new_ts = ts_out[None]
    mask = jnp.zeros((1, 1, 1, 1025), dtype=hidden_states.dtype)
    return preserved, mask, new_ts


# speedup vs baseline: 1.0738x; 1.0738x over previous
import functools

import jax
import jax.numpy as jnp
from jax import lax
from jax.experimental import pallas as pl
from jax.experimental.pallas import tpu as pltpu
from jax.experimental.pallas import tpu_sc as plsc

_DEF = jax.lax.Precision.DEFAULT
_HI = jax.lax.Precision.HIGHEST


def _tc_body(kl_ref, ts_ref, t_ref, rcp_ref, tso_ref):
    mt = jnp.mean(kl_ref[...], axis=0)
    m = jnp.transpose(mt, (1, 0))
    m = m / jnp.sqrt(jnp.sum(m * m, axis=1, keepdims=True))
    ci = jax.lax.broadcasted_iota(jnp.int32, (1024, 2048), 1)
    rk = jax.lax.broadcasted_iota(jnp.int32, (1024, 1), 0)
    so = jnp.where(ci == 2 * rk + 1, 1.0, 0.0)
    b = jax.lax.dot_general(so, m, (((1,), (0,)), ((), ())),
                            precision=_HI)
    scores = jax.lax.dot_general(m, b, (((1,), (1,)), ((), ())),
                                 precision=_DEF)
    node_max = jnp.max(scores, axis=1, keepdims=True)
    ck = jax.lax.broadcasted_iota(jnp.int32, (2048, 1024), 1)
    nidx = jnp.min(jnp.where(scores == node_max, ck, 1024),
                   axis=1, keepdims=True)
    ri = jax.lax.broadcasted_iota(jnp.int32, (2048, 1), 0)
    evenok = (ri % 2 == 0) & (ri >= 2)
    merge = evenok & (ck == nidx)
    is_dst = ri == 2 * ck + 1
    mm = jnp.where(merge | is_dst, 1.0, 0.0)
    ones = jnp.ones((2048, 1), dtype=jnp.float32)
    cnt1 = jax.lax.dot_general(mm, ones, (((0,), (0,)), ((), ())),
                               precision=_DEF)
    ts_y = jax.lax.dot_general(mm, ts_ref[...], (((0,), (0,)), ((), ())),
                               precision=_DEF)
    odd = ri % 2 == 1
    t_ref[...] = jnp.where(odd, 1 + (ri - 1) // 2,
                           jnp.where(ri == 0, 1025, 1 + nidx))
    rcp_ref[...] = 1.0 / cnt1
    tso_ref[0:1, :] = ts_ref[0:1, :]
    tso_ref[pl.ds(1, 1024), :] = ts_y


def _sc_body(hid_hbm, idx_hbm, rcp_hbm, out_hbm, idx_v, rcp_v, src_v, acc):
    c = lax.axis_index("c")
    s = lax.axis_index("s")
    t = s * 2 + c

    @pl.when(t < 24)
    def _():
        tc = t // 4
        lc0 = 32 * (t % 4)
        c0 = t * 32
        pltpu.sync_copy(idx_hbm, idx_v)
        pltpu.sync_copy(rcp_hbm, rcp_v)
        pltpu.sync_copy(hid_hbm.at[:, tc, :, pl.ds(lc0, 32)], src_v)

        acc[0, pl.ds(0, 16)] = src_v[0, 0, pl.ds(0, 16)]
        acc[0, pl.ds(16, 16)] = src_v[0, 0, pl.ds(16, 16)]

        def _dst(g, _):
            for r in range(1, 16, 2):
                row = 1 + 8 * g + (r - 1) // 2
                tr = 2 * g + r // 8
                sr = r % 8
                acc[row, pl.ds(0, 16)] = src_v[tr, sr, pl.ds(0, 16)]
                acc[row, pl.ds(16, 16)] = src_v[tr, sr, pl.ds(16, 16)]
            return 0

        lax.fori_loop(0, 128, _dst, 0)

        def _accum(g, _):
            idx16 = idx_v[pl.ds(g * 16, 16)]
            for r in range(0, 16, 2):
                d = idx16[r]
                tr = 2 * g + r // 8
                sr = r % 8
                acc[d, pl.ds(0, 16)] = (
                    acc[d, pl.ds(0, 16)] + src_v[tr, sr, pl.ds(0, 16)])
                acc[d, pl.ds(16, 16)] = (
                    acc[d, pl.ds(16, 16)] + src_v[tr, sr, pl.ds(16, 16)])
            return 0

        lax.fori_loop(0, 128, _accum, 0)

        def _scale(g, _):
            w16 = rcp_v[pl.ds(g * 16, 16)]
            for r in range(16):
                row = 1 + g * 16 + r
                w = jnp.full((16,), w16[r], dtype=jnp.float32)
                acc[row, pl.ds(0, 16)] = acc[row, pl.ds(0, 16)] * w
                acc[row, pl.ds(16, 16)] = acc[row, pl.ds(16, 16)] * w
            return 0

        lax.fori_loop(0, 64, _scale, 0)
        pltpu.sync_copy(acc.at[pl.ds(0, 1025)], out_hbm.at[:, pl.ds(c0, 32)])


_sc_merge = functools.partial(
    pl.kernel,
    out_type=jax.ShapeDtypeStruct((1025, 768), jnp.float32),
    mesh=plsc.VectorSubcoreMesh(core_axis_name="c", subcore_axis_name="s"),
    compiler_params=pltpu.CompilerParams(use_tc_tiling_on_sc=False),
    scratch_types=[
        pltpu.VMEM((2048,), jnp.int32),
        pltpu.VMEM((1024,), jnp.float32),
        pltpu.VMEM((256, 8, 32), jnp.float32),
        pltpu.VMEM((1026, 32), jnp.float32),
    ],
)(_sc_body)


def kernel(hidden_states, attention_mask, self_attention_scores, key_layer,
           tome_size):
    del attention_mask, self_attention_scores
    tgt, rcp, ts_out = pl.pallas_call(
        _tc_body,
        out_shape=(
            jax.ShapeDtypeStruct((2048, 1), jnp.int32),
            jax.ShapeDtypeStruct((1024, 1), jnp.float32),
            jax.ShapeDtypeStruct((1025, 1), jnp.float32),
        ),
    )(jnp.transpose(key_layer[0], (0, 2, 1)), tome_size[0])

    hid4 = jnp.transpose(hidden_states[0].reshape(256, 8, 6, 128),
                         (0, 2, 1, 3))
    out = _sc_merge(hid4, tgt.reshape(2048), rcp.reshape(1024))

    preserved = out[None]
    new_ts = ts_out[None]
    mask = jnp.zeros((1, 1, 1, 1025), dtype=hidden_states.dtype)
    return preserved, mask, new_ts
